# baseline (speedup 1.0000x reference)
import math

import jax
import jax.numpy as jnp
import numpy as np
from jax.experimental import pallas as pl
from jax.experimental.pallas import tpu as pltpu

_H = 32
_W = 32
_C = 96
_C8 = 8 * _C
_P = _H * _W


def _build_maps():
    center = (_H - 1) / 2.0
    learnable = []
    for i in range(_H):
        for j in range(_W):
            y = center - i
            x = j - center
            if x == 0 and y == 0:
                learnable.append((i, j))
            else:
                ang = math.atan2(y, x)
                if ang < 0:
                    ang += 2 * math.pi
                if 0 <= ang <= math.pi / 4 + 1e-06:
                    learnable.append((i, j))
    src = -np.ones(_P, dtype=np.int64)
    rot = np.zeros(_P, dtype=np.int64)
    for idx, (i, j) in enumerate(learnable):
        for k in range(8):
            y = center - i
            x = j - center
            theta = k * math.pi / 4
            cos_t = math.cos(theta)
            sin_t = math.sin(theta)
            x_new = cos_t * x - sin_t * y
            y_new = sin_t * x + cos_t * y
            i_r = int(round(center - y_new))
            j_r = int(round(center + x_new))
            i_r = max(0, min(_H - 1, i_r))
            j_r = max(0, min(_W - 1, j_r))
            p = i_r * _W + j_r
            src[p] = idx
            rot[p] = k
    mask = src >= 0
    src = np.where(mask, src, 0)
    return len(learnable), src, rot, mask


_NL, _SRC, _ROT, _MASK = _build_maps()

_ONEHOT = np.zeros((_P, _NL), dtype=np.float32)
_ONEHOT[np.arange(_P), _SRC] = 1.0
_ROTMASKS = np.stack(
    [((_ROT == k) & _MASK).astype(np.float32) for k in range(8)], axis=0
)


def _grid_body(onehot_ref, masks_ref, pe_ref, cls_ref, g_ref):
    rows = jnp.dot(
        onehot_ref[...], pe_ref[...], preferred_element_type=jnp.float32
    )
    acc = jnp.zeros((_P, _C8), jnp.float32)
    for k in range(8):
        s = ((8 - k) % 8) * _C
        if s:
            rolled = jnp.concatenate([rows[:, s:], rows[:, :s]], axis=1)
        else:
            rolled = rows
        acc = acc + masks_ref[k, :][:, None] * rolled
    g_ref[0:1, :] = jnp.concatenate([cls_ref[...]] * 8, axis=1)
    g_ref[1:, :] = acc


_GROWS = 6150
_XROWS = 64 * _GROWS


def _add_body(g_ref, x_ref, o_ref):
    for i in range(4):
        sl = pl.ds(i * _GROWS, _GROWS)
        o_ref[sl, :] = x_ref[sl, :] + g_ref[...]


@jax.jit
def kernel(x, pos_embed_learnable, cls_pos_eighth):
    B = x.shape[0]
    pe = pos_embed_learnable[0]
    cls = cls_pos_eighth[0]
    onehot = jnp.asarray(_ONEHOT)
    masks = jnp.asarray(_ROTMASKS)
    full_grid = pl.pallas_call(
        _grid_body,
        out_shape=jax.ShapeDtypeStruct((1 + _P, _C8), jnp.float32),
    )(onehot, masks, pe, cls)
    xf = x.reshape(_XROWS, 128)
    g6 = full_grid.reshape(_GROWS, 128)
    out = pl.pallas_call(
        _add_body,
        grid=(16,),
        in_specs=[
            pl.BlockSpec((_GROWS, 128), lambda i: (0, 0)),
            pl.BlockSpec((4 * _GROWS, 128), lambda i: (i, 0)),
        ],
        out_specs=pl.BlockSpec((4 * _GROWS, 128), lambda i: (i, 0)),
        out_shape=jax.ShapeDtypeStruct((_XROWS, 128), x.dtype),
    )(g6, xf)
    return out.reshape(x.shape)


# speedup vs baseline: 1.8910x; 1.8910x over previous
import math

import jax
import jax.numpy as jnp
import numpy as np
from jax.experimental import pallas as pl
from jax.experimental.pallas import tpu as pltpu

_H = 32
_W = 32
_C = 96
_C8 = 8 * _C
_P = _H * _W


def _build_maps():
    center = (_H - 1) / 2.0
    learnable = []
    for i in range(_H):
        for j in range(_W):
            y = center - i
            x = j - center
            if x == 0 and y == 0:
                learnable.append((i, j))
            else:
                ang = math.atan2(y, x)
                if ang < 0:
                    ang += 2 * math.pi
                if 0 <= ang <= math.pi / 4 + 1e-06:
                    learnable.append((i, j))
    src = -np.ones(_P, dtype=np.int64)
    rot = np.zeros(_P, dtype=np.int64)
    for idx, (i, j) in enumerate(learnable):
        for k in range(8):
            y = center - i
            x = j - center
            theta = k * math.pi / 4
            cos_t = math.cos(theta)
            sin_t = math.sin(theta)
            x_new = cos_t * x - sin_t * y
            y_new = sin_t * x + cos_t * y
            i_r = int(round(center - y_new))
            j_r = int(round(center + x_new))
            i_r = max(0, min(_H - 1, i_r))
            j_r = max(0, min(_W - 1, j_r))
            p = i_r * _W + j_r
            src[p] = idx
            rot[p] = k
    mask = src >= 0
    src = np.where(mask, src, 0)
    return len(learnable), src, rot, mask


_NL, _SRC, _ROT, _MASK = _build_maps()

_ONEHOT = np.zeros((_P, _NL), dtype=np.float32)
_ONEHOT[np.arange(_P), _SRC] = 1.0
_ROTMASKS = np.stack(
    [((_ROT == k) & _MASK).astype(np.float32) for k in range(8)], axis=0
)


def _grid_body(onehot_ref, masks_ref, pe_ref, cls_ref, g_ref):
    rows = jnp.dot(
        onehot_ref[...], pe_ref[...], preferred_element_type=jnp.float32
    )
    acc = jnp.zeros((_P, _C8), jnp.float32)
    for k in range(8):
        s = ((8 - k) % 8) * _C
        if s:
            rolled = jnp.concatenate([rows[:, s:], rows[:, :s]], axis=1)
        else:
            rolled = rows
        acc = acc + masks_ref[k, :][:, None] * rolled
    g_ref[0:1, :] = jnp.concatenate([cls_ref[...]] * 8, axis=1)
    g_ref[1:, :] = acc


_NBUF = 8
_HC = _C8 // 2


def _add_body(g_ref, x_hbm, o_hbm, xbuf, obuf, sin, sout):
    B = x_hbm.shape[0]
    nch = 2 * B

    def in_copy(c, slot):
        b, h = c // 2, c % 2
        return pltpu.make_async_copy(
            x_hbm.at[b, :, pl.ds(h * _HC, _HC)], xbuf.at[slot], sin.at[slot]
        )

    def out_copy(c, slot):
        b, h = c // 2, c % 2
        return pltpu.make_async_copy(
            obuf.at[slot], o_hbm.at[b, :, pl.ds(h * _HC, _HC)], sout.at[slot]
        )

    for c in range(_NBUF):
        in_copy(c, c).start(priority=c % 2)
    for c in range(nch):
        slot = c % _NBUF
        in_copy(c, slot).wait()
        if c >= _NBUF:
            out_copy(c - _NBUF, slot).wait()
        obuf[slot] = xbuf[slot] + g_ref[:, pl.ds((c % 2) * _HC, _HC)]
        out_copy(c, slot).start(priority=c % 2)
        if c + _NBUF < nch:
            in_copy(c + _NBUF, slot).start(priority=c % 2)
    for c in range(nch - _NBUF, nch):
        out_copy(c, c % _NBUF).wait()


@jax.jit
def kernel(x, pos_embed_learnable, cls_pos_eighth):
    B = x.shape[0]
    pe = pos_embed_learnable[0]
    cls = cls_pos_eighth[0]
    onehot = jnp.asarray(_ONEHOT)
    masks = jnp.asarray(_ROTMASKS)
    full_grid = pl.pallas_call(
        _grid_body,
        out_shape=jax.ShapeDtypeStruct((1 + _P, _C8), jnp.float32),
    )(onehot, masks, pe, cls)
    out = pl.pallas_call(
        _add_body,
        in_specs=[
            pl.BlockSpec((1 + _P, _C8), lambda: (0, 0)),
            pl.BlockSpec(memory_space=pltpu.HBM),
        ],
        out_specs=pl.BlockSpec(memory_space=pltpu.HBM),
        out_shape=jax.ShapeDtypeStruct(x.shape, x.dtype),
        scratch_shapes=[
            pltpu.VMEM((_NBUF, 1 + _P, _HC), jnp.float32),
            pltpu.VMEM((_NBUF, 1 + _P, _HC), jnp.float32),
            pltpu.SemaphoreType.DMA((_NBUF,)),
            pltpu.SemaphoreType.DMA((_NBUF,)),
        ],
    )(full_grid, x)
    return out


# speedup vs baseline: 6.0721x; 3.2111x over previous
import math

import jax
import jax.numpy as jnp
import numpy as np
from jax.experimental import pallas as pl
from jax.experimental.pallas import tpu as pltpu

_H = 32
_W = 32
_C = 96
_C8 = 8 * _C
_P = _H * _W


def _build_maps():
    center = (_H - 1) / 2.0
    learnable = []
    for i in range(_H):
        for j in range(_W):
            y = center - i
            x = j - center
            if x == 0 and y == 0:
                learnable.append((i, j))
            else:
                ang = math.atan2(y, x)
                if ang < 0:
                    ang += 2 * math.pi
                if 0 <= ang <= math.pi / 4 + 1e-06:
                    learnable.append((i, j))
    src = -np.ones(_P, dtype=np.int64)
    rot = np.zeros(_P, dtype=np.int64)
    for idx, (i, j) in enumerate(learnable):
        for k in range(8):
            y = center - i
            x = j - center
            theta = k * math.pi / 4
            cos_t = math.cos(theta)
            sin_t = math.sin(theta)
            x_new = cos_t * x - sin_t * y
            y_new = sin_t * x + cos_t * y
            i_r = int(round(center - y_new))
            j_r = int(round(center + x_new))
            i_r = max(0, min(_H - 1, i_r))
            j_r = max(0, min(_W - 1, j_r))
            p = i_r * _W + j_r
            src[p] = idx
            rot[p] = k
    mask = src >= 0
    src = np.where(mask, src, 0)
    return len(learnable), src, rot, mask


_NL, _SRC, _ROT, _MASK = _build_maps()

_ONEHOT = np.zeros((_P, _NL), dtype=np.float32)
_ONEHOT[np.arange(_P), _SRC] = 1.0
_ROTMASKS = np.stack(
    [((_ROT == k) & _MASK).astype(np.float32) for k in range(8)], axis=0
)


def _grid_body(onehot_ref, masks_ref, pe_ref, cls_ref, g_ref):
    rows = jnp.dot(
        onehot_ref[...], pe_ref[...], preferred_element_type=jnp.float32
    )
    acc = jnp.zeros((_P, _C8), jnp.float32)
    for k in range(8):
        s = ((8 - k) % 8) * _C
        if s:
            rolled = jnp.concatenate([rows[:, s:], rows[:, :s]], axis=1)
        else:
            rolled = rows
        acc = acc + masks_ref[k, :][:, None] * rolled
    g_ref[0:1, :] = jnp.concatenate([cls_ref[...]] * 8, axis=1)
    g_ref[1:, :] = acc


_RBLK = 32


def _add_body(g_ref, x_ref, o_ref):
    o_ref[...] = x_ref[...] + g_ref[...][:, None, :]


@jax.jit
def kernel(x, pos_embed_learnable, cls_pos_eighth):
    B = x.shape[0]
    pe = pos_embed_learnable[0]
    cls = cls_pos_eighth[0]
    onehot = jnp.asarray(_ONEHOT)
    masks = jnp.asarray(_ROTMASKS)
    full_grid = pl.pallas_call(
        _grid_body,
        out_shape=jax.ShapeDtypeStruct((1 + _P, _C8), jnp.float32),
    )(onehot, masks, pe, cls)
    xt = jnp.transpose(x, (1, 0, 2))
    nblk = (1 + _P + _RBLK - 1) // _RBLK
    out_t = pl.pallas_call(
        _add_body,
        grid=(nblk,),
        in_specs=[
            pl.BlockSpec((_RBLK, _C8), lambda i: (i, 0)),
            pl.BlockSpec((_RBLK, B, _C8), lambda i: (i, 0, 0)),
        ],
        out_specs=pl.BlockSpec((_RBLK, B, _C8), lambda i: (i, 0, 0)),
        out_shape=jax.ShapeDtypeStruct(xt.shape, x.dtype),
    )(full_grid, xt)
    return jnp.transpose(out_t, (1, 0, 2))
